# Initial kernel scaffold; baseline (speedup 1.0000x reference)
import jax
import jax.numpy as jnp
from jax import lax
from jax.experimental import pallas as pl
from jax.experimental.pallas import tpu as pltpu
from jax.experimental.pallas import tpu_sc as plsc

N = 10000
E = 320000
D = 128
W = 144
NTILES = 32
EPT = E // NTILES
CH = 125
NCH = EPT // CH
RPT = N // 16


def _pre_body(x_ref, beta_ref, inv_ref, invb_ref):
    xb = x_ref[...]
    n2 = jnp.sum(xb * xb, axis=1, keepdims=True)
    inv = 1.0 / jnp.maximum(jnp.sqrt(n2), 1e-12)
    inv_ref[...] = inv
    invb_ref[...] = inv * beta_ref[0, 0]


def _post_body(p_ref, o_ref):
    acc = p_ref[0] + p_ref[1]
    feat = acc[:, :D]
    den = acc[:, D:D + 1]
    o_ref[...] = jnp.maximum(feat / jnp.maximum(den, 1e-16), 0.0)


def _sc_kernel_body(x_hbm, src_hbm, dst_hbm, inv_hbm, invb_hbm, num_hbm,
                    src_v, dst_v, inv_v, invb_v, xs_v, xd_v, o_v, num_sh, sem):
    core = lax.axis_index("c")
    sub = lax.axis_index("s")
    wid = sub * 2 + core

    pltpu.sync_copy(src_hbm.at[wid], src_v)
    pltpu.sync_copy(dst_hbm.at[wid], dst_v)
    pltpu.sync_copy(inv_hbm, inv_v)
    pltpu.sync_copy(invb_hbm, invb_v)

    zeros16 = jnp.zeros((16,), jnp.float32)

    def zero_row(r, _):
        for k in range(W // 16):
            o_v[r, pl.ds(16 * k, 16)] = zeros16
        return 0

    lax.fori_loop(0, CH, zero_row, 0)

    for j in range(RPT // CH):
        pltpu.sync_copy(o_v, num_sh.at[pl.ds(sub * RPT + j * CH, CH)])

    plsc.subcore_barrier()

    lane = lax.broadcasted_iota(jnp.int32, (16,), 0)

    def do_chunk(ci, _):
        g1 = pltpu.async_copy(x_hbm.at[src_v.at[ci]], xs_v, sem)
        g2 = pltpu.async_copy(x_hbm.at[dst_v.at[ci]], xd_v, sem)
        g1.wait()
        g2.wait()

        def do_edge(e, _2):
            si = src_v[ci, e]
            di = dst_v[ci, e]
            a0 = xs_v[e, pl.ds(0, 16)]
            b0 = xd_v[e, pl.ds(0, 16)]
            acc = a0 * b0
            rows_a = [a0]
            for k in range(1, D // 16):
                ak = xs_v[e, pl.ds(16 * k, 16)]
                bk = xd_v[e, pl.ds(16 * k, 16)]
                rows_a.append(ak)
                acc = acc + ak * bk
            dot = jnp.sum(acc)
            alpha = dot * inv_v[si] * invb_v[di]
            ex = jnp.exp(jnp.full((16,), alpha, jnp.float32))
            for k in range(D // 16):
                o_v[e, pl.ds(16 * k, 16)] = rows_a[k] * ex
            o_v[e, pl.ds(D, 16)] = jnp.where(lane == 0, ex, 0.0)
            return 0

        lax.fori_loop(0, CH, do_edge, 0)
        pltpu.sync_copy(o_v, num_sh.at[dst_v.at[ci]], add=True)
        return 0

    lax.fori_loop(0, NCH, do_chunk, 0)

    plsc.subcore_barrier()

    pltpu.sync_copy(num_sh.at[pl.ds(sub * RPT, RPT)],
                    num_hbm.at[core, pl.ds(sub * RPT, RPT)])


@jax.jit
def kernel(x, edge_index, beta):
    src = edge_index[0].reshape(NTILES, NCH, CH)
    dst = edge_index[1].reshape(NTILES, NCH, CH)

    inv, invb = pl.pallas_call(
        _pre_body,
        out_shape=[
            jax.ShapeDtypeStruct((N, 1), jnp.float32),
            jax.ShapeDtypeStruct((N, 1), jnp.float32),
        ],
        in_specs=[
            pl.BlockSpec(memory_space=pltpu.VMEM),
            pl.BlockSpec(memory_space=pltpu.SMEM),
        ],
    )(x, beta.reshape(1, 1))
    inv = inv.reshape(N)
    invb = invb.reshape(N)

    mesh = plsc.VectorSubcoreMesh(core_axis_name="c", subcore_axis_name="s")
    num = pl.kernel(
        _sc_kernel_body,
        out_type=jax.ShapeDtypeStruct((2, N, W), jnp.float32),
        mesh=mesh,
        scratch_types=[
            pltpu.VMEM((NCH, CH), jnp.int32),
            pltpu.VMEM((NCH, CH), jnp.int32),
            pltpu.VMEM((N,), jnp.float32),
            pltpu.VMEM((N,), jnp.float32),
            pltpu.VMEM((CH, D), jnp.float32),
            pltpu.VMEM((CH, D), jnp.float32),
            pltpu.VMEM((CH, W), jnp.float32),
            pltpu.VMEM_SHARED((N, W), jnp.float32),
            pltpu.SemaphoreType.DMA,
        ],
    )(x, src, dst, inv, invb)

    out = pl.pallas_call(
        _post_body,
        grid=(10,),
        out_shape=jax.ShapeDtypeStruct((N, D), jnp.float32),
        in_specs=[pl.BlockSpec((2, N // 10, W), lambda i: (0, i, 0))],
        out_specs=pl.BlockSpec((N // 10, D), lambda i: (i, 0)),
    )(num)
    return out


# speedup vs baseline: 2.9487x; 2.9487x over previous
import jax
import jax.numpy as jnp
from jax import lax
from jax.experimental import pallas as pl
from jax.experimental.pallas import tpu as pltpu
from jax.experimental.pallas import tpu_sc as plsc

N = 10000
E = 320000
D = 128
W = 144
TN = 10256
HALF = 5120
NACC = 5136
NTILES = 32
EPT = E // NTILES
CH = 80
CAPL = 5760
CAPB = CAPL + 16
NLCH = CAPL // CH


def _pre_body(x_ref, beta_ref, t_ref):
    xb = x_ref[...]
    n2 = jnp.sum(xb * xb, axis=1, keepdims=True)
    inv = 1.0 / jnp.maximum(jnp.sqrt(n2), 1e-12)
    t_ref[:, :D] = xb
    t_ref[:, D:D + 1] = inv
    t_ref[:, D + 1:D + 2] = inv * beta_ref[0, 0]
    t_ref[:, D + 2:] = jnp.zeros((x_ref.shape[0], W - D - 2), jnp.float32)


def _post_body(p_ref, o_ref):
    b = p_ref[0]
    feat = b[:, :D]
    den = b[:, D:D + 1]
    o_ref[...] = jnp.maximum(feat / jnp.maximum(den, 1e-16), 0.0)


def _route_body(src_hbm, dst_hbm, ls_hbm, ld_hbm, hs_hbm, hd_hbm,
                sv_v, dv_v, ls_v, ld_v, hs_v, hd_v):
    core = lax.axis_index("c")
    sub = lax.axis_index("s")
    wid = sub * 2 + core

    pltpu.sync_copy(src_hbm.at[wid], sv_v)
    pltpu.sync_copy(dst_hbm.at[wid], dv_v)

    zs = jnp.zeros((16,), jnp.int32)
    pad_low = jnp.full((16,), HALF + 7, jnp.int32)
    pad_high = jnp.full((16,), TN - 1, jnp.int32)

    def prefill(r, _):
        sl = pl.ds(16 * r, 16)
        ls_v[sl] = zs
        ld_v[sl] = pad_low
        hs_v[sl] = zs
        hd_v[sl] = pad_high
        return 0

    lax.fori_loop(0, CAPB // 16, prefill, 0)

    def grp(g, carry):
        p_lo, p_hi = carry
        sl = pl.ds(16 * g, 16)
        sv = sv_v[sl]
        dv = dv_v[sl]
        ml = dv < HALF
        mh = dv >= HALF
        cl = plsc.all_reduce_population_count(ml)[0]

        @pl.when(p_lo <= CAPL)
        def _():
            plsc.store_compressed(ls_v.at[pl.ds(p_lo, 16)], sv, mask=ml)
            plsc.store_compressed(ld_v.at[pl.ds(p_lo, 16)], dv, mask=ml)

        @pl.when(p_hi <= CAPL)
        def _():
            plsc.store_compressed(hs_v.at[pl.ds(p_hi, 16)], sv, mask=mh)
            plsc.store_compressed(hd_v.at[pl.ds(p_hi, 16)], dv, mask=mh)

        return p_lo + cl, p_hi + (16 - cl)

    lax.fori_loop(0, EPT // 16, grp, (jnp.int32(0), jnp.int32(0)))

    pltpu.sync_copy(ls_v.at[pl.ds(0, CAPL)], ls_hbm.at[wid])
    pltpu.sync_copy(ld_v.at[pl.ds(0, CAPL)], ld_hbm.at[wid])
    pltpu.sync_copy(hs_v.at[pl.ds(0, CAPL)], hs_hbm.at[wid])
    pltpu.sync_copy(hd_v.at[pl.ds(0, CAPL)], hd_hbm.at[wid])


def _main_body(t_hbm, ls_hbm, ld_hbm, hs_hbm, hd_hbm, acc_hbm,
               sl_v, dl_v, dloc_v, xs_v, xd_v, o_v, acc_sh, sem):
    core = lax.axis_index("c")
    sub = lax.axis_index("s")
    s2 = sub * 2

    @pl.when(core == 0)
    def _():
        pltpu.sync_copy(ls_hbm.at[s2], sl_v.at[0])
        pltpu.sync_copy(ls_hbm.at[s2 + 1], sl_v.at[1])
        pltpu.sync_copy(ld_hbm.at[s2], dl_v.at[0])
        pltpu.sync_copy(ld_hbm.at[s2 + 1], dl_v.at[1])

    @pl.when(core == 1)
    def _():
        pltpu.sync_copy(hs_hbm.at[s2], sl_v.at[0])
        pltpu.sync_copy(hs_hbm.at[s2 + 1], sl_v.at[1])
        pltpu.sync_copy(hd_hbm.at[s2], dl_v.at[0])
        pltpu.sync_copy(hd_hbm.at[s2 + 1], dl_v.at[1])

    zeros16 = jnp.zeros((16,), jnp.float32)

    def zero_row(r, _):
        for k in range(W // 16):
            o_v[r, pl.ds(16 * k, 16)] = zeros16
        return 0

    lax.fori_loop(0, CH, zero_row, 0)

    base = sub * 320
    for j in range(4):
        pltpu.sync_copy(o_v, acc_sh.at[pl.ds(base + j * CH, CH)])

    @pl.when(sub == 15)
    def _():
        pltpu.sync_copy(o_v.at[pl.ds(0, 16)], acc_sh.at[pl.ds(HALF, 16)])

    plsc.subcore_barrier()

    lane = lax.broadcasted_iota(jnp.int32, (16,), 0)
    off = core * HALF

    def do_chunk(i, _):
        li = i // NLCH
        e0 = (i % NLCH) * CH
        g1 = pltpu.async_copy(t_hbm.at[sl_v.at[li, pl.ds(e0, CH)]], xs_v, sem)
        g2 = pltpu.async_copy(t_hbm.at[dl_v.at[li, pl.ds(e0, CH)]], xd_v, sem)
        for k in range(CH // 16):
            dloc_v[pl.ds(16 * k, 16)] = dl_v[li, pl.ds(e0 + 16 * k, 16)] - off
        g1.wait()
        g2.wait()

        def do_edge(e, _2):
            a0 = xs_v[e, pl.ds(0, 16)]
            b0 = xd_v[e, pl.ds(0, 16)]
            acc = a0 * b0
            rows_a = [a0]
            for k in range(1, D // 16):
                ak = xs_v[e, pl.ds(16 * k, 16)]
                bk = xd_v[e, pl.ds(16 * k, 16)]
                rows_a.append(ak)
                acc = acc + ak * bk
            dot = jnp.sum(acc)
            xs_ex = xs_v[e, pl.ds(D, 16)]
            xd_ex = xd_v[e, pl.ds(D, 16)]
            alpha = dot * xs_ex[0] * xd_ex[1]
            ex = jnp.exp(jnp.full((16,), alpha, jnp.float32))
            for k in range(D // 16):
                o_v[e, pl.ds(16 * k, 16)] = rows_a[k] * ex
            o_v[e, pl.ds(D, 16)] = jnp.where(lane == 0, ex, 0.0)
            return 0

        lax.fori_loop(0, CH, do_edge, 0)
        pltpu.sync_copy(o_v, acc_sh.at[dloc_v], add=True)
        return 0

    lax.fori_loop(0, 2 * NLCH, do_chunk, 0)

    plsc.subcore_barrier()

    pltpu.sync_copy(acc_sh.at[pl.ds(base, 320)],
                    acc_hbm.at[core, pl.ds(base, 320)])

    @pl.when(sub == 15)
    def _():
        pltpu.sync_copy(acc_sh.at[pl.ds(HALF, 16)],
                        acc_hbm.at[core, pl.ds(HALF, 16)])


@jax.jit
def kernel(x, edge_index, beta):
    src = edge_index[0].reshape(NTILES, EPT)
    dst = edge_index[1].reshape(NTILES, EPT)
    x_pad = jnp.zeros((TN, D), jnp.float32).at[:N].set(x)

    t = pl.pallas_call(
        _pre_body,
        out_shape=jax.ShapeDtypeStruct((TN, W), jnp.float32),
        in_specs=[
            pl.BlockSpec(memory_space=pltpu.VMEM),
            pl.BlockSpec(memory_space=pltpu.SMEM),
        ],
    )(x_pad, beta.reshape(1, 1))

    mesh = plsc.VectorSubcoreMesh(core_axis_name="c", subcore_axis_name="s")
    sc_params = pltpu.CompilerParams(
        needs_layout_passes=False, use_tc_tiling_on_sc=False)

    ls, ld, hs, hd = pl.kernel(
        _route_body,
        out_type=[jax.ShapeDtypeStruct((NTILES, CAPL), jnp.int32)] * 4,
        mesh=mesh,
        compiler_params=sc_params,
        scratch_types=[
            pltpu.VMEM((EPT,), jnp.int32),
            pltpu.VMEM((EPT,), jnp.int32),
            pltpu.VMEM((CAPB,), jnp.int32),
            pltpu.VMEM((CAPB,), jnp.int32),
            pltpu.VMEM((CAPB,), jnp.int32),
            pltpu.VMEM((CAPB,), jnp.int32),
        ],
    )(src, dst)

    acc = pl.kernel(
        _main_body,
        out_type=jax.ShapeDtypeStruct((2, NACC, W), jnp.float32),
        mesh=mesh,
        compiler_params=sc_params,
        scratch_types=[
            pltpu.VMEM((2, CAPL), jnp.int32),
            pltpu.VMEM((2, CAPL), jnp.int32),
            pltpu.VMEM((CH,), jnp.int32),
            pltpu.VMEM((CH, W), jnp.float32),
            pltpu.VMEM((CH, W), jnp.float32),
            pltpu.VMEM((CH, W), jnp.float32),
            pltpu.VMEM_SHARED((NACC, W), jnp.float32),
            pltpu.SemaphoreType.DMA,
        ],
    )(t, ls, ld, hs, hd)

    def lo_map(j):
        c = jnp.where(j < 64, 0, 1)
        return c, jnp.where(j < 64, j, j - 64), 0

    out = pl.pallas_call(
        _post_body,
        grid=(N // CH,),
        out_shape=jax.ShapeDtypeStruct((N, D), jnp.float32),
        in_specs=[pl.BlockSpec((1, CH, W), lo_map)],
        out_specs=pl.BlockSpec((CH, D), lambda j: (j, 0)),
    )(acc)
    return out
